# baseline (speedup 1.0000x reference)
import functools

import jax
import jax.numpy as jnp
from jax import lax
from jax.experimental import pallas as pl
from jax.experimental.pallas import tpu as pltpu
from jax.experimental.pallas import tpu_sc as plsc

_NUM_CLASSES = 100000
_SPECIAL_ROW = _NUM_CLASSES + 1
_N = 16384
_D = 64
_LANES = 16


@functools.lru_cache(maxsize=None)
def _make_lookup():
    info = plsc.get_sparse_core_info()
    nw = info.num_cores * info.num_subcores
    bpw = _N // nw
    mesh = plsc.VectorSubcoreMesh(core_axis_name="c", subcore_axis_name="s")

    @functools.partial(
        pl.kernel,
        mesh=mesh,
        out_type=jax.ShapeDtypeStruct((_N, _D), jnp.float32),
        scratch_types=[
            pltpu.VMEM((bpw,), jnp.int32),
            pltpu.VMEM((bpw, _D), jnp.float32),
            pltpu.SemaphoreType.DMA,
        ],
        compiler_params=pltpu.CompilerParams(use_tc_tiling_on_sc=False),
    )
    def lookup(labels_hbm, table_hbm, out_hbm, idx_v, rows_v, sem):
        wid = lax.axis_index("s") * info.num_cores + lax.axis_index("c")
        base = wid * bpw
        pltpu.sync_copy(labels_hbm.at[pl.ds(base, bpw)], idx_v)
        for i in range(bpw // _LANES):
            sl = pl.ds(i * _LANES, _LANES)
            v = idx_v[sl]
            idx_v[sl] = jnp.where(v < 0, _SPECIAL_ROW, v)
        pltpu.async_copy(table_hbm.at[idx_v], rows_v, sem).wait()
        pltpu.sync_copy(rows_v, out_hbm.at[pl.ds(base, bpw)])

    return lookup


def kernel(labels, train, embedding_table):
    if labels.ndim == 0:
        labels = labels[None]
    lookup = _make_lookup()
    return lookup(labels, embedding_table)
